# baseline (speedup 1.0000x reference)
import functools

import jax
import jax.numpy as jnp
from jax import lax
from jax.experimental import pallas as pl
from jax.experimental.pallas import tpu as pltpu
from jax.experimental.pallas import tpu_sc as plsc

BATCH = 4096
SEQ = 200
HIDDEN = 64
PAD = 2 * HIDDEN
NUM_TOKENS = BATCH * SEQ
NUM_WORKERS = 32
B_PER_W = BATCH // NUM_WORKERS
TOK_W = B_PER_W * SEQ
CH = 128
GT = 512
CPG = GT // CH
NG = TOK_W // GT


@jax.jit
def _embed(tokens, table):
    mesh = plsc.VectorSubcoreMesh(core_axis_name="c", subcore_axis_name="s")

    @functools.partial(
        pl.kernel,
        mesh=mesh,
        compiler_params=pltpu.CompilerParams(use_tc_tiling_on_sc=False),
        out_type=jax.ShapeDtypeStruct((NUM_TOKENS, PAD), jnp.float32),
        scratch_types=[
            pltpu.VMEM((TOK_W,), jnp.int32),
            pltpu.VMEM((GT, HIDDEN), jnp.float32),
            pltpu.VMEM((GT, HIDDEN), jnp.float32),
            pltpu.SemaphoreType.DMA,
            pltpu.SemaphoreType.DMA,
            pltpu.SemaphoreType.DMA,
            pltpu.SemaphoreType.DMA,
        ],
    )
    def k(tok_hbm, table_hbm, out_hbm, idx_v, buf_a, buf_b, gsem_a, gsem_b,
          osem_a, osem_b):
        wid = lax.axis_index("s") * 2 + lax.axis_index("c")
        t0 = wid * TOK_W
        pltpu.sync_copy(tok_hbm.at[pl.ds(t0, TOK_W)], idx_v)

        def fire(g, buf, gsem):
            for j in range(CPG):
                pltpu.async_copy(
                    table_hbm.at[idx_v.at[pl.ds(g * GT + j * CH, CH)]],
                    buf.at[pl.ds(j * CH, CH)],
                    gsem,
                )

        def drain(buf, gsem):
            for j in range(CPG):
                pltpu.make_async_copy(
                    table_hbm.at[idx_v.at[pl.ds(0, CH)]],
                    buf.at[pl.ds(j * CH, CH)],
                    gsem,
                ).wait()

        def out_slice(g):
            return out_hbm.at[pl.ds(t0 + g * GT, GT), pl.ds(0, HIDDEN)]

        def store(g, buf, osem):
            pltpu.async_copy(buf, out_slice(g), osem)

        def store_wait(g, buf, osem):
            pltpu.make_async_copy(buf, out_slice(g), osem).wait()

        fire(0, buf_a, gsem_a)
        fire(1, buf_b, gsem_b)
        drain(buf_a, gsem_a)
        store(0, buf_a, osem_a)

        def body(i, carry):
            store_wait(2 * i, buf_a, osem_a)
            fire(2 * i + 2, buf_a, gsem_a)
            drain(buf_b, gsem_b)
            store(2 * i + 1, buf_b, osem_b)
            store_wait(2 * i + 1, buf_b, osem_b)
            fire(2 * i + 3, buf_b, gsem_b)
            drain(buf_a, gsem_a)
            store(2 * i + 2, buf_a, osem_a)
            return carry

        lax.fori_loop(0, (NG - 2) // 2, body, 0)

        drain(buf_b, gsem_b)
        store(NG - 1, buf_b, osem_b)
        store_wait(NG - 2, buf_a, osem_a)
        store_wait(NG - 1, buf_b, osem_b)

    return k(tokens, table)


def kernel(tokens, embedding_weight):
    out = _embed(tokens.astype(jnp.int32).ravel(), embedding_weight)
    return out.reshape(BATCH, SEQ, PAD)[..., :HIDDEN]
